# Initial kernel scaffold; baseline (speedup 1.0000x reference)
import jax
import jax.numpy as jnp
from jax.experimental import pallas as pl


def kernel(f, weight, bias):
    raise NotImplementedError("write your pallas kernel here")


# speedup vs baseline: 1.0019x; 1.0019x over previous
import functools

import jax
import jax.numpy as jnp
from jax.experimental import pallas as pl


def _matmul_body(f_ref, w_ref, b_ref, o_ref):
    k = pl.program_id(1)
    acc = jnp.dot(f_ref[...], w_ref[0], preferred_element_type=jnp.float32)

    @pl.when(k == 0)
    def _init():
        o_ref[...] = acc + b_ref[...]

    @pl.when(k != 0)
    def _accum():
        o_ref[...] += acc


@functools.partial(jax.jit, static_argnames=())
def kernel(f, weight, bias):
    B, F = f.shape
    L, _, D = weight.shape
    KBLK = 2048
    nk = F // KBLK

    out = pl.pallas_call(
        _matmul_body,
        grid=(L, nk),
        in_specs=[
            pl.BlockSpec((B, KBLK), lambda l, k: (0, k)),
            pl.BlockSpec((1, KBLK, D), lambda l, k: (l, k, 0)),
            pl.BlockSpec((1, D), lambda l, k: (0, l)),
        ],
        out_specs=pl.BlockSpec((B, D), lambda l, k: (0, l)),
        out_shape=jax.ShapeDtypeStruct((B, L * D), jnp.float32),
    )(f, weight, bias.reshape(1, L * D))
    return out.reshape(B, L, D)


# speedup vs baseline: 1.0177x; 1.0157x over previous
import functools

import jax
import jax.numpy as jnp
from jax.experimental import pallas as pl


def _matmul_body(f_ref, w_ref, b_ref, o_ref):
    k = pl.program_id(0)
    L = w_ref.shape[0]
    D = w_ref.shape[2]
    for l in range(L):
        acc = jnp.dot(f_ref[...], w_ref[l], preferred_element_type=jnp.float32)
        col = pl.ds(l * D, D)

        @pl.when(k == 0)
        def _init():
            o_ref[:, col] = acc + b_ref[:, col]

        @pl.when(k != 0)
        def _accum():
            o_ref[:, col] += acc


@functools.partial(jax.jit, static_argnames=())
def kernel(f, weight, bias):
    B, F = f.shape
    L, _, D = weight.shape
    KBLK = 2048
    nk = F // KBLK

    out = pl.pallas_call(
        _matmul_body,
        grid=(nk,),
        in_specs=[
            pl.BlockSpec((B, KBLK), lambda k: (0, k)),
            pl.BlockSpec((L, KBLK, D), lambda k: (0, k, 0)),
            pl.BlockSpec((1, L * D), lambda k: (0, 0)),
        ],
        out_specs=pl.BlockSpec((B, L * D), lambda k: (0, 0)),
        out_shape=jax.ShapeDtypeStruct((B, L * D), jnp.float32),
    )(f, weight, bias.reshape(1, L * D))
    return out.reshape(B, L, D)
